# Initial kernel scaffold; baseline (speedup 1.0000x reference)
import jax
import jax.numpy as jnp
from jax.experimental import pallas as pl


def kernel(tensor_in, pos_table):
    raise NotImplementedError("write your pallas kernel here")


# speedup vs baseline: 2.6364x; 2.6364x over previous
import jax
import jax.numpy as jnp
from jax.experimental import pallas as pl


_SEQ_BLOCK = 512


def _add_block(tensor_ref, pos_ref, out_ref):
    out_ref[...] = tensor_ref[...] + pos_ref[...]


def kernel(tensor_in, pos_table):
    batch, seq, dim = tensor_in.shape
    grid = (seq // _SEQ_BLOCK, batch)
    return pl.pallas_call(
        _add_block,
        grid=grid,
        in_specs=[
            pl.BlockSpec((1, _SEQ_BLOCK, dim), lambda i, j: (j, i, 0)),
            pl.BlockSpec((_SEQ_BLOCK, dim), lambda i, j: (i, 0)),
        ],
        out_specs=pl.BlockSpec((1, _SEQ_BLOCK, dim), lambda i, j: (j, i, 0)),
        out_shape=jax.ShapeDtypeStruct(tensor_in.shape, tensor_in.dtype),
    )(tensor_in, pos_table)


# speedup vs baseline: 2.7986x; 1.0615x over previous
import jax
import jax.numpy as jnp
from jax.experimental import pallas as pl


_SEQ_BLOCK = 1024


def _add_block(tensor_ref, pos_ref, out_ref):
    out_ref[...] = tensor_ref[...] + pos_ref[...]


def kernel(tensor_in, pos_table):
    batch, seq, dim = tensor_in.shape
    grid = (seq // _SEQ_BLOCK, batch)
    return pl.pallas_call(
        _add_block,
        grid=grid,
        in_specs=[
            pl.BlockSpec((1, _SEQ_BLOCK, dim), lambda i, j: (j, i, 0)),
            pl.BlockSpec((_SEQ_BLOCK, dim), lambda i, j: (i, 0)),
        ],
        out_specs=pl.BlockSpec((1, _SEQ_BLOCK, dim), lambda i, j: (j, i, 0)),
        out_shape=jax.ShapeDtypeStruct(tensor_in.shape, tensor_in.dtype),
    )(tensor_in, pos_table)
